# baseline (speedup 1.0000x reference)
import functools

import jax
import jax.numpy as jnp
from jax import lax
from jax.experimental import pallas as pl
from jax.experimental.pallas import tpu as pltpu
from jax.experimental.pallas import tpu_sc as plsc

_NE = 1024
_NTOK = 16 * 4096
_LANES = 16


def _sc_partial_hist(flat_idx, nc, ns):
  nw = nc * ns
  chunk = _NTOK // nw
  mesh = plsc.VectorSubcoreMesh(
      core_axis_name="c", subcore_axis_name="s", num_cores=nc)

  @functools.partial(
      pl.kernel,
      out_type=jax.ShapeDtypeStruct((nw, _NE), jnp.float32),
      mesh=mesh,
      compiler_params=pltpu.CompilerParams(
          needs_layout_passes=False,
          disable_bounds_checks=True,
          disable_semaphore_checks=True,
      ),
      scratch_types=[
          pltpu.VMEM((chunk,), jnp.int32),
          pltpu.VMEM((_NE,), jnp.float32),
          pltpu.SemaphoreType.DMA,
      ],
  )
  def hist(idx_hbm, out_hbm, idx_v, counts_v, sem):
    wid = lax.axis_index("s") * nc + lax.axis_index("c")
    base = wid * chunk
    cp = pltpu.async_copy(idx_hbm.at[pl.ds(base, chunk)], idx_v, sem)

    zeros = jnp.zeros((_LANES,), jnp.float32)

    def zero_body(i, carry):
      counts_v[pl.ds(i * _LANES, _LANES)] = zeros
      return carry

    lax.fori_loop(0, _NE // _LANES, zero_body, 0, unroll=8)
    cp.wait()

    ones = jnp.ones((_LANES,), jnp.float32)

    def body(i, carry):
      idx = idx_v[pl.ds(i * _LANES, _LANES)]
      plsc.addupdate_scatter(counts_v, [idx], ones)
      return carry

    lax.fori_loop(0, chunk // _LANES, body, 0, unroll=8)

    pltpu.sync_copy(counts_v, out_hbm.at[wid])

  return hist(flat_idx)


def _finish_body(vq_ref, ne_ref, p_ref, *out_ref):
  p = p_ref[...]
  counts = jnp.sum(p, axis=0, keepdims=True)
  usage = counts * (1.0 / _NTOK)
  ent = -jnp.sum(usage * jnp.log(usage + 1e-08))
  util = jnp.mean((usage > 1e-06).astype(jnp.float32))
  ne = ne_ref[...].astype(jnp.float32)
  max_ent = jnp.sum(jnp.log(jnp.full((1, 128), ne, jnp.float32))) * (
      1.0 / 128.0)
  commit = 0.25 * vq_ref[...]
  div = -0.1 * (ent / max_ent)
  t_ref, c_ref, d_ref, u_ref = out_ref
  t_ref[...] = commit + div
  c_ref[...] = commit
  d_ref[...] = div
  u_ref[...] = util


def kernel(vq_loss, indices, num_embeddings):
  nc, ns = 1, 16
  flat = indices.reshape(-1)
  partials = _sc_partial_hist(flat, nc, ns)

  vq = jnp.asarray(vq_loss, jnp.float32)
  ne = jnp.asarray(num_embeddings, jnp.int32)
  out = pl.pallas_call(
      _finish_body,
      compiler_params=pltpu.CompilerParams(
          disable_bounds_checks=True,
          allow_input_fusion=[True, True, False],
      ),
      out_shape=[jax.ShapeDtypeStruct((), jnp.float32)] * 4,
      in_specs=[
          pl.BlockSpec(memory_space=pltpu.SMEM),
          pl.BlockSpec(memory_space=pltpu.SMEM),
          pl.BlockSpec(memory_space=pltpu.VMEM),
      ],
      out_specs=[pl.BlockSpec(memory_space=pltpu.SMEM)] * 4,
  )(vq, ne, partials)
  return (out[0], out[1], out[2], out[3])


# speedup vs baseline: 1.0037x; 1.0037x over previous
import functools

import jax
import jax.numpy as jnp
from jax import lax
from jax.experimental import pallas as pl
from jax.experimental.pallas import tpu as pltpu
from jax.experimental.pallas import tpu_sc as plsc

_NE = 1024
_NTOK = 16 * 4096
_LANES = 16


def _sc_partial_hist(flat_idx, nc, ns):
  nw = nc * ns
  chunk = _NTOK // nw
  mesh = plsc.VectorSubcoreMesh(
      core_axis_name="c", subcore_axis_name="s", num_cores=nc)

  @functools.partial(
      pl.kernel,
      out_type=jax.ShapeDtypeStruct((nw, _NE), jnp.float32),
      mesh=mesh,
      compiler_params=pltpu.CompilerParams(
          needs_layout_passes=False,
          disable_bounds_checks=True,
          disable_semaphore_checks=True,
      ),
      scratch_types=[
          pltpu.VMEM((chunk,), jnp.int32),
          pltpu.VMEM((_NE,), jnp.float32),
          pltpu.VMEM((_NE,), jnp.float32),
          pltpu.SemaphoreType.DMA,
      ],
  )
  def hist(idx_hbm, out_hbm, idx_v, counts_v, counts_b, sem):
    wid = lax.axis_index("s") * nc + lax.axis_index("c")
    base = wid * chunk
    cp = pltpu.async_copy(idx_hbm.at[pl.ds(base, chunk)], idx_v, sem)

    zeros = jnp.zeros((_LANES,), jnp.float32)

    def zero_body(i, carry):
      counts_v[pl.ds(i * _LANES, _LANES)] = zeros
      counts_b[pl.ds(i * _LANES, _LANES)] = zeros
      return carry

    lax.fori_loop(0, _NE // _LANES, zero_body, 0, unroll=8)
    cp.wait()

    ones = jnp.ones((_LANES,), jnp.float32)

    def body(i, carry):
      ia = idx_v[pl.ds(i * 2 * _LANES, _LANES)]
      ib = idx_v[pl.ds(i * 2 * _LANES + _LANES, _LANES)]
      plsc.addupdate_scatter(counts_v, [ia], ones)
      plsc.addupdate_scatter(counts_b, [ib], ones)
      return carry

    lax.fori_loop(0, chunk // (2 * _LANES), body, 0, unroll=8)

    def merge_body(i, carry):
      sl = pl.ds(i * _LANES, _LANES)
      counts_v[sl] = counts_v[sl] + counts_b[sl]
      return carry

    lax.fori_loop(0, _NE // _LANES, merge_body, 0, unroll=8)

    pltpu.sync_copy(counts_v, out_hbm.at[wid])

  return hist(flat_idx)


def _finish_body(vq_ref, ne_ref, p_ref, *out_ref):
  p = p_ref[...]
  counts = jnp.sum(p, axis=0, keepdims=True)
  usage = counts * (1.0 / _NTOK)
  ent = -jnp.sum(usage * jnp.log(usage + 1e-08))
  util = jnp.mean((usage > 1e-06).astype(jnp.float32))
  ne = ne_ref[...].astype(jnp.float32)
  max_ent = jnp.sum(jnp.log(jnp.full((1, 128), ne, jnp.float32))) * (
      1.0 / 128.0)
  commit = 0.25 * vq_ref[...]
  div = -0.1 * (ent / max_ent)
  t_ref, c_ref, d_ref, u_ref = out_ref
  t_ref[...] = commit + div
  c_ref[...] = commit
  d_ref[...] = div
  u_ref[...] = util


def kernel(vq_loss, indices, num_embeddings):
  nc, ns = 1, 16
  flat = indices.reshape(-1)
  partials = _sc_partial_hist(flat, nc, ns)

  vq = jnp.asarray(vq_loss, jnp.float32)
  ne = jnp.asarray(num_embeddings, jnp.int32)
  out = pl.pallas_call(
      _finish_body,
      compiler_params=pltpu.CompilerParams(
          disable_bounds_checks=True,
          allow_input_fusion=[True, True, False],
      ),
      out_shape=[jax.ShapeDtypeStruct((), jnp.float32)] * 4,
      in_specs=[
          pl.BlockSpec(memory_space=pltpu.SMEM),
          pl.BlockSpec(memory_space=pltpu.SMEM),
          pl.BlockSpec(memory_space=pltpu.VMEM),
      ],
      out_specs=[pl.BlockSpec(memory_space=pltpu.SMEM)] * 4,
  )(vq, ne, partials)
  return (out[0], out[1], out[2], out[3])
